# baseline (speedup 1.0000x reference)
import functools

import jax
import jax.numpy as jnp
from jax import lax
from jax.experimental import pallas as pl
from jax.experimental.pallas import tpu as pltpu
from jax.experimental.pallas import tpu_sc as plsc

V = 1000001
D = 64
T = 20
B = 16384
CPS = D // 2
BPT = B // 16
GPT = BPT // 128

VBLK = 8192
NVB = -(-V // VBLK)
CSTRIDE = NVB * VBLK
SLEN = 1000064

_mesh = plsc.VectorSubcoreMesh(core_axis_name="c", subcore_axis_name="s")


VMAIN = 999936


@functools.partial(
    pl.pallas_call,
    grid=(D // 8,),
    in_specs=[
        pl.BlockSpec(memory_space=pl.ANY),
        pl.BlockSpec((8, 128), lambda cb: (cb, VMAIN // 128)),
    ],
    out_specs=[pl.BlockSpec(memory_space=pl.ANY) for _ in range(8)],
    out_shape=[
        jax.ShapeDtypeStruct((D // 8 * CSTRIDE,), jnp.float32) for _ in range(8)
    ],
    scratch_shapes=[pltpu.SemaphoreType.DMA, pltpu.SemaphoreType.DMA],
)
def _detile(wt_ref, tail_ref, *rest):
    o_refs, sem, sem2 = rest[:8], rest[8], rest[9]
    cb = pl.program_id(0)

    def row_copy(k):
        return pltpu.make_async_copy(
            wt_ref.at[cb * 8 + k, pl.ds(0, VMAIN)],
            o_refs[k].at[pl.ds(cb * CSTRIDE, VMAIN)],
            sem,
        )

    def tail_copy(k):
        return pltpu.make_async_copy(
            tail_ref.at[k, :],
            o_refs[k].at[pl.ds(cb * CSTRIDE + VMAIN, 128)],
            sem2,
        )

    for k in range(8):
        row_copy(k).start()
        tail_copy(k).start()
    for k in range(8):
        row_copy(k).wait()
        tail_copy(k).wait()


@functools.partial(
    pl.kernel,
    mesh=_mesh,
    out_type=jax.ShapeDtypeStruct((T, D, B), jnp.float32),
    compiler_params=pltpu.CompilerParams(use_tc_tiling_on_sc=False),
    scratch_types=[
        pltpu.VMEM_SHARED((2, SLEN), jnp.float32),
        pltpu.VMEM((2, GPT, 128), jnp.int32),
        pltpu.VMEM((2, BPT), jnp.float32),
        pltpu.SemaphoreType.DMA,
        pltpu.SemaphoreType.DMA,
        pltpu.SemaphoreType.DMA,
        pltpu.SemaphoreType.DMA,
    ],
)
def _emb_gather(w0, w1, w2, w3, w4, w5, w6, w7, idx3, o,
                sp, idx_v, out_v, ssem, isem, gsem, osem):
    wts = (w0, w1, w2, w3, w4, w5, w6, w7)
    sc = lax.axis_index("c")
    w = lax.axis_index("s")
    b0 = w * BPT
    c_base = sc * CPS

    def idx_dma(i, slot):
        t = i % T
        return pltpu.make_async_copy(
            idx3.at[t, pl.ds(w * GPT, GPT), :], idx_v.at[slot], isem
        )

    def out_dma(i, slot):
        ci = i // T
        t = i % T
        return pltpu.make_async_copy(
            out_v.at[slot], o.at[t, c_base + ci, pl.ds(b0, BPT)], osem
        )

    def stage_dma_k(ci, slot, k):
        off = (sc * (CPS // 8) + ci // 8) * CSTRIDE
        return pltpu.make_async_copy(
            wts[k].at[pl.ds(off, SLEN)], sp.at[slot], ssem
        )

    def stage_start(ci, slot):
        for k in range(8):
            @pl.when(ci % 8 == k)
            def _():
                stage_dma_k(ci, slot, k).start()

    def stage_wait(ci, slot):
        for k in range(8):
            @pl.when(ci % 8 == k)
            def _():
                stage_dma_k(ci, slot, k).wait()

    @pl.when(w == 0)
    def _():
        stage_dma_k(0, 0, 0).start()

    idx_dma(0, 0).start()

    def body(i, carry):
        ci = i // T
        t = i % T
        sbuf = ci % 2
        slot = i % 2

        @pl.when(t == 0)
        def _():
            @pl.when(w == 0)
            def _():
                stage_wait(ci, sbuf)

            plsc.subcore_barrier()

            @pl.when((w == 0) & (ci + 1 < CPS))
            def _():
                stage_start(ci + 1, 1 - sbuf)

        @pl.when(i + 1 < CPS * T)
        def _():
            idx_dma(i + 1, 1 - slot).start()

        idx_dma(i, slot).wait()

        @pl.when(i >= 2)
        def _():
            out_dma(i - 2, slot).wait()

        for j in range(GPT):
            pltpu.async_copy(
                sp.at[sbuf].at[idx_v.at[slot, j]],
                out_v.at[slot, pl.ds(j * 128, 128)],
                gsem,
            )
        for j in range(GPT):
            pltpu.make_async_copy(
                sp.at[sbuf].at[idx_v.at[slot, j]],
                out_v.at[slot, pl.ds(j * 128, 128)],
                gsem,
            ).wait()

        out_dma(i, slot).start()
        return carry

    n = CPS * T
    lax.fori_loop(0, n, body, 0)
    out_dma(n - 2, n % 2).wait()
    out_dma(n - 1, (n - 1) % 2).wait()


def kernel(input, weight):
    wt = weight.T
    wts = _detile(wt, wt)
    idx3 = input.T.reshape(T, B // 128, 128)
    o = _emb_gather(*wts, idx3)
    return jnp.transpose(o, (2, 0, 1))


# speedup vs baseline: 10.7809x; 10.7809x over previous
import functools

import jax
import jax.numpy as jnp
from jax import lax
from jax.experimental import pallas as pl
from jax.experimental.pallas import tpu as pltpu
from jax.experimental.pallas import tpu_sc as plsc

V = 1000001
VP = 1000008
D = 64
T = 20
B = 16384
N = T * B
NW = 32
RPW = N // (128 * NW)

_mesh = plsc.VectorSubcoreMesh(core_axis_name="c", subcore_axis_name="s")


@functools.partial(
    pl.kernel,
    mesh=_mesh,
    out_type=jax.ShapeDtypeStruct((N, 128), jnp.float32),
    compiler_params=pltpu.CompilerParams(use_tc_tiling_on_sc=False),
    scratch_types=[
        pltpu.VMEM((2, 128), jnp.int32),
        pltpu.VMEM((2, 128, 128), jnp.float32),
        pltpu.SemaphoreType.DMA,
        pltpu.SemaphoreType.DMA,
        pltpu.SemaphoreType.DMA,
        pltpu.SemaphoreType.DMA,
        pltpu.SemaphoreType.DMA,
        pltpu.SemaphoreType.DMA,
    ],
)
def _emb_gather(wpad, idx2, o, idx_v, rows_v,
                isem0, isem1, gsem0, gsem1, osem0, osem1):
    isems = (isem0, isem1)
    gsems = (gsem0, gsem1)
    osems = (osem0, osem1)
    sc = lax.axis_index("c")
    w = lax.axis_index("s") * 2 + sc
    r0 = w * RPW

    def idx_dma(s, u):
        return pltpu.make_async_copy(idx2.at[r0 + s, :], idx_v.at[u], isems[u])

    def gather(u):
        return pltpu.make_async_copy(
            wpad.at[idx_v.at[u]], rows_v.at[u], gsems[u]
        )

    def out_dma(s, u):
        return pltpu.make_async_copy(
            rows_v.at[u],
            o.at[pl.ds((r0 + s) * 128, 128), :],
            osems[u],
        )

    idx_dma(0, 0).start()

    def body(g, carry):
        for u in (0, 1):
            s = 2 * g + u

            @pl.when(s + 1 < RPW)
            def _():
                idx_dma(s + 1, 1 - u).start()

            idx_dma(s, u).wait()

            @pl.when(s >= 1)
            def _():
                gather(1 - u).wait()
                out_dma(s - 1, 1 - u).start()

            @pl.when(s >= 2)
            def _():
                out_dma(s - 2, u).wait()

            gather(u).start()
        return carry

    lax.fori_loop(0, RPW // 2, body, 0)
    gather(1).wait()
    out_dma(RPW - 1, 1).start()
    out_dma(RPW - 2, 0).wait()
    out_dma(RPW - 1, 1).wait()


def kernel(input, weight):
    wpad = jnp.pad(weight, ((0, VP - V), (0, 128 - D)))
    idx2 = input.T.reshape(N // 128, 128)
    o = _emb_gather(wpad, idx2)
    return jnp.transpose(o[:, :D].reshape(T, B, D), (1, 0, 2))
